# baseline (speedup 1.0000x reference)
import jax
import jax.numpy as jnp
from jax import lax
from jax.experimental import pallas as pl
from jax.experimental.pallas import tpu as pltpu
from jax.experimental.pallas import tpu_sc as plsc

M = 65536
D = 64
B = 16
H = 512
W = 512
NC = 2
NS = 16
L = 16
NW = NC * NS
ROWS_PER_W = (B * H) // NW
CHUNK_ROWS = 16
NCHUNK = ROWS_PER_W // CHUNK_ROWS
VECS_PER_ROW = W // L


def _hist_body(rm_hbm, counts_hbm, hist, buf0, buf1, sem0, sem1):
    c = lax.axis_index("c")
    s = lax.axis_index("s")
    wid = c * NS + s

    zeros = jnp.zeros((L,), jnp.float32)
    ones = jnp.ones((L,), jnp.float32)

    img = wid // NC
    half = wid % NC
    row0 = half * ROWS_PER_W

    def start_dma(chunk_idx, buf, sem):
        r = row0 + chunk_idx * CHUNK_ROWS
        return pltpu.make_async_copy(
            rm_hbm.at[img, pl.ds(r, CHUNK_ROWS), :], buf, sem)

    start_dma(0, buf0, sem0).start()

    ZU = 64

    def zero_body(i, _):
        for u in range(ZU):
            hist[pl.ds((i * ZU + u) * L, L)] = zeros
        return 0
    lax.fori_loop(0, M // (L * ZU), zero_body, 0, unroll=False)

    def process(buf):
        @plsc.parallel_loop(0, CHUNK_ROWS, 1, unroll=2)
        def row_body(r):
            vecs = [buf[r, pl.ds(j * L, L)] for j in range(VECS_PER_ROW)]
            for vec in vecs:
                plsc.addupdate_scatter(hist, [vec], ones)

    def pair_body(p, _):
        k = p * 2
        start_dma(k + 1, buf1, sem1).start()
        start_dma(0, buf0, sem0).wait()
        process(buf0)
        nxt = jnp.minimum(k + 2, NCHUNK - 1)
        start_dma(nxt, buf0, sem0).start()
        start_dma(0, buf1, sem1).wait()
        process(buf1)
        return 0
    lax.fori_loop(0, NCHUNK // 2, pair_body, 0, unroll=False)
    start_dma(0, buf0, sem0).wait()

    pltpu.sync_copy(hist, counts_hbm.at[wid])


def _make_hist_kernel():
    mesh = plsc.VectorSubcoreMesh(core_axis_name="c", subcore_axis_name="s")
    return pl.kernel(
        _hist_body,
        out_type=jax.ShapeDtypeStruct((NW, M), jnp.float32),
        mesh=mesh,
        compiler_params=pltpu.CompilerParams(needs_layout_passes=False),
        scratch_types=[
            pltpu.VMEM((M,), jnp.float32),
            pltpu.VMEM((CHUNK_ROWS, W), jnp.int32),
            pltpu.VMEM((CHUNK_ROWS, W), jnp.int32),
            pltpu.SemaphoreType.DMA,
            pltpu.SemaphoreType.DMA,
        ],
    )


def _scale_body(counts_ref, tableT_ref, outT_ref):
    csum = jnp.sum(counts_ref[...], axis=0)
    outT_ref[...] = tableT_ref[...] * csum[None, :]


BINS_PER_BLK = 8192


def _scale(table, counts):
    tableT = table.T
    grid = (M // BINS_PER_BLK,)
    outT = pl.pallas_call(
        _scale_body,
        grid=grid,
        in_specs=[
            pl.BlockSpec((NW, BINS_PER_BLK), lambda i: (0, i)),
            pl.BlockSpec((D, BINS_PER_BLK), lambda i: (0, i)),
        ],
        out_specs=pl.BlockSpec((D, BINS_PER_BLK), lambda i: (0, i)),
        out_shape=jax.ShapeDtypeStruct((D, M), jnp.float32),
    )(counts, tableT)
    return outT.T


@jax.jit
def kernel(region_attention_table, region_map):
    counts = _make_hist_kernel()(region_map)
    return _scale(region_attention_table, counts)


# speedup vs baseline: 1.0144x; 1.0144x over previous
import jax
import jax.numpy as jnp
from jax import lax
from jax.experimental import pallas as pl
from jax.experimental.pallas import tpu as pltpu
from jax.experimental.pallas import tpu_sc as plsc

M = 65536
D = 64
B = 16
H = 512
W = 512
NC = 2
NS = 16
L = 16
NW = NC * NS
ROWS_PER_W = (B * H) // NW
CHUNK_ROWS = 32
NCHUNK = ROWS_PER_W // CHUNK_ROWS
VECS_PER_ROW = W // L


def _hist_body(rm_hbm, counts_hbm, hist, buf0, buf1, sem0, sem1):
    c = lax.axis_index("c")
    s = lax.axis_index("s")
    wid = c * NS + s

    zeros = jnp.zeros((L,), jnp.float32)
    ones = jnp.ones((L,), jnp.float32)

    img = wid // NC
    half = wid % NC
    row0 = half * ROWS_PER_W

    def start_dma(chunk_idx, buf, sem):
        r = row0 + chunk_idx * CHUNK_ROWS
        return pltpu.make_async_copy(
            rm_hbm.at[img, pl.ds(r, CHUNK_ROWS), :], buf, sem)

    start_dma(0, buf0, sem0).start()

    ZU = 64

    def zero_body(i, _):
        for u in range(ZU):
            hist[pl.ds((i * ZU + u) * L, L)] = zeros
        return 0
    lax.fori_loop(0, M // (L * ZU), zero_body, 0, unroll=False)

    def process(buf):
        def row_body(r, _):
            vecs = [buf[r, pl.ds(j * L, L)] for j in range(VECS_PER_ROW)]
            for vec in vecs:
                plsc.addupdate_scatter(hist, [vec], ones)
            return 0
        lax.fori_loop(0, CHUNK_ROWS, row_body, 0, unroll=False)

    def pair_body(p, _):
        k = p * 2
        start_dma(k + 1, buf1, sem1).start()
        start_dma(0, buf0, sem0).wait()
        process(buf0)
        nxt = jnp.minimum(k + 2, NCHUNK - 1)
        start_dma(nxt, buf0, sem0).start()
        start_dma(0, buf1, sem1).wait()
        process(buf1)
        return 0
    lax.fori_loop(0, NCHUNK // 2, pair_body, 0, unroll=False)
    start_dma(0, buf0, sem0).wait()

    pltpu.sync_copy(hist, counts_hbm.at[wid])


def _make_hist_kernel():
    mesh = plsc.VectorSubcoreMesh(core_axis_name="c", subcore_axis_name="s")
    return pl.kernel(
        _hist_body,
        out_type=jax.ShapeDtypeStruct((NW, M), jnp.float32),
        mesh=mesh,
        compiler_params=pltpu.CompilerParams(needs_layout_passes=False),
        scratch_types=[
            pltpu.VMEM((M,), jnp.float32),
            pltpu.VMEM((CHUNK_ROWS, W), jnp.int32),
            pltpu.VMEM((CHUNK_ROWS, W), jnp.int32),
            pltpu.SemaphoreType.DMA,
            pltpu.SemaphoreType.DMA,
        ],
    )


def _scale_body(counts_ref, tableT_ref, outT_ref):
    csum = jnp.sum(counts_ref[...], axis=0)
    outT_ref[...] = tableT_ref[...] * csum[None, :]


BINS_PER_BLK = 8192


def _scale(table, counts):
    tableT = table.T
    grid = (M // BINS_PER_BLK,)
    outT = pl.pallas_call(
        _scale_body,
        grid=grid,
        in_specs=[
            pl.BlockSpec((NW, BINS_PER_BLK), lambda i: (0, i)),
            pl.BlockSpec((D, BINS_PER_BLK), lambda i: (0, i)),
        ],
        out_specs=pl.BlockSpec((D, BINS_PER_BLK), lambda i: (0, i)),
        out_shape=jax.ShapeDtypeStruct((D, M), jnp.float32),
    )(counts, tableT)
    return outT.T


@jax.jit
def kernel(region_attention_table, region_map):
    counts = _make_hist_kernel()(region_map)
    return _scale(region_attention_table, counts)


# speedup vs baseline: 1.0345x; 1.0197x over previous
import jax
import jax.numpy as jnp
from jax import lax
from jax.experimental import pallas as pl
from jax.experimental.pallas import tpu as pltpu
from jax.experimental.pallas import tpu_sc as plsc

M = 65536
D = 64
B = 16
H = 512
W = 512
NC = 2
NS = 16
L = 16
NW = NC * NS
ROWS_PER_W = (B * H) // NW
CHUNK_ROWS = 32
NCHUNK = ROWS_PER_W // CHUNK_ROWS
VECS_PER_ROW = W // L


def _hist_body(rm_hbm, counts_hbm, hist, buf0, buf1, sem0, sem1):
    c = lax.axis_index("c")
    s = lax.axis_index("s")
    wid = c * NS + s

    zeros = jnp.zeros((L,), jnp.float32)
    ones = jnp.ones((L,), jnp.float32)

    img = wid // NC
    half = wid % NC
    row0 = half * ROWS_PER_W

    def start_dma(chunk_idx, buf, sem):
        r = row0 + chunk_idx * CHUNK_ROWS
        return pltpu.make_async_copy(
            rm_hbm.at[img, pl.ds(r, CHUNK_ROWS), :], buf, sem)

    start_dma(0, buf0, sem0).start()

    ZU = 64

    def zero_body(i, _):
        for u in range(ZU):
            hist[pl.ds((i * ZU + u) * L, L)] = zeros
        return 0
    lax.fori_loop(0, M // (L * ZU), zero_body, 0, unroll=False)

    def process(buf):
        def row_body(r, _):
            vecs = [buf[r, pl.ds(j * L, L)] for j in range(VECS_PER_ROW)]
            for vec in vecs:
                plsc.addupdate_scatter(hist, [vec], ones)
            return 0
        lax.fori_loop(0, CHUNK_ROWS, row_body, 0, unroll=False)

    def pair_body(p, _):
        k = p * 2
        start_dma(k + 1, buf1, sem1).start()
        start_dma(0, buf0, sem0).wait()
        process(buf0)
        nxt = jnp.minimum(k + 2, NCHUNK - 1)
        start_dma(nxt, buf0, sem0).start()
        start_dma(0, buf1, sem1).wait()
        process(buf1)
        return 0
    lax.fori_loop(0, NCHUNK // 2, pair_body, 0, unroll=False)
    start_dma(0, buf0, sem0).wait()

    pltpu.sync_copy(hist, counts_hbm.at[wid])


def _make_hist_kernel():
    mesh = plsc.VectorSubcoreMesh(core_axis_name="c", subcore_axis_name="s")
    return pl.kernel(
        _hist_body,
        out_type=jax.ShapeDtypeStruct((NW, M), jnp.float32),
        mesh=mesh,
        compiler_params=pltpu.CompilerParams(needs_layout_passes=False),
        scratch_types=[
            pltpu.VMEM((M,), jnp.float32),
            pltpu.VMEM((CHUNK_ROWS, W), jnp.int32),
            pltpu.VMEM((CHUNK_ROWS, W), jnp.int32),
            pltpu.SemaphoreType.DMA,
            pltpu.SemaphoreType.DMA,
        ],
    )


def _scale_body(counts_ref, tableT_ref, outT_ref):
    csum = jnp.sum(counts_ref[...], axis=0)
    outT_ref[...] = tableT_ref[...] * csum[None, :]


BINS_PER_BLK = 16384


def _scale(table, counts):
    tableT = table.T
    grid = (M // BINS_PER_BLK,)
    outT = pl.pallas_call(
        _scale_body,
        grid=grid,
        in_specs=[
            pl.BlockSpec((NW, BINS_PER_BLK), lambda i: (0, i)),
            pl.BlockSpec((D, BINS_PER_BLK), lambda i: (0, i)),
        ],
        out_specs=pl.BlockSpec((D, BINS_PER_BLK), lambda i: (0, i)),
        out_shape=jax.ShapeDtypeStruct((D, M), jnp.float32),
    )(counts, tableT)
    return outT.T


@jax.jit
def kernel(region_attention_table, region_map):
    counts = _make_hist_kernel()(region_map)
    return _scale(region_attention_table, counts)


# speedup vs baseline: 1.0370x; 1.0025x over previous
import jax
import jax.numpy as jnp
from jax import lax
from jax.experimental import pallas as pl
from jax.experimental.pallas import tpu as pltpu
from jax.experimental.pallas import tpu_sc as plsc

M = 65536
D = 64
B = 16
H = 512
W = 512
NC = 2
NS = 16
L = 16
NW = NC * NS
ROWS_PER_W = (B * H) // NW
CHUNK_ROWS = 32
NCHUNK = ROWS_PER_W // CHUNK_ROWS
VECS_PER_ROW = W // L


def _hist_body(rm_hbm, counts_hbm, hist, buf0, buf1, sem0, sem1):
    c = lax.axis_index("c")
    s = lax.axis_index("s")
    wid = c * NS + s

    zeros = jnp.zeros((L,), jnp.float32)
    ones = jnp.ones((L,), jnp.float32)

    img = wid // NC
    half = wid % NC
    row0 = half * ROWS_PER_W

    def start_dma(chunk_idx, buf, sem):
        r = row0 + chunk_idx * CHUNK_ROWS
        return pltpu.make_async_copy(
            rm_hbm.at[img, pl.ds(r, CHUNK_ROWS), :], buf, sem)

    start_dma(0, buf0, sem0).start()

    ZU = 64

    def zero_body(i, _):
        for u in range(ZU):
            hist[pl.ds((i * ZU + u) * L, L)] = zeros
        return 0
    lax.fori_loop(0, M // (L * ZU), zero_body, 0, unroll=False)

    def process(buf):
        def row_body(r, _):
            vecs = [buf[r, pl.ds(j * L, L)] for j in range(VECS_PER_ROW)]
            for vec in vecs:
                plsc.addupdate_scatter(hist, [vec], ones)
            return 0
        lax.fori_loop(0, CHUNK_ROWS, row_body, 0, unroll=False)

    def pair_body(p, _):
        k = p * 2
        start_dma(k + 1, buf1, sem1).start()
        start_dma(0, buf0, sem0).wait()
        process(buf0)
        nxt = jnp.minimum(k + 2, NCHUNK - 1)
        start_dma(nxt, buf0, sem0).start()
        start_dma(0, buf1, sem1).wait()
        process(buf1)
        return 0
    lax.fori_loop(0, NCHUNK // 2, pair_body, 0, unroll=False)
    start_dma(0, buf0, sem0).wait()

    pltpu.sync_copy(hist, counts_hbm.at[wid])


def _make_hist_kernel():
    mesh = plsc.VectorSubcoreMesh(core_axis_name="c", subcore_axis_name="s")
    return pl.kernel(
        _hist_body,
        out_type=jax.ShapeDtypeStruct((NW, M), jnp.float32),
        mesh=mesh,
        compiler_params=pltpu.CompilerParams(needs_layout_passes=False),
        scratch_types=[
            pltpu.VMEM((M,), jnp.float32),
            pltpu.VMEM((CHUNK_ROWS, W), jnp.int32),
            pltpu.VMEM((CHUNK_ROWS, W), jnp.int32),
            pltpu.SemaphoreType.DMA,
            pltpu.SemaphoreType.DMA,
        ],
    )


def _scale_body(counts_ref, tableT_ref, outT_ref):
    csum = jnp.sum(counts_ref[...], axis=0)
    outT_ref[...] = tableT_ref[...] * csum[None, :]


BINS_PER_BLK = 32768


def _scale(table, counts):
    tableT = table.T
    grid = (M // BINS_PER_BLK,)
    outT = pl.pallas_call(
        _scale_body,
        grid=grid,
        in_specs=[
            pl.BlockSpec((NW, BINS_PER_BLK), lambda i: (0, i)),
            pl.BlockSpec((D, BINS_PER_BLK), lambda i: (0, i)),
        ],
        out_specs=pl.BlockSpec((D, BINS_PER_BLK), lambda i: (0, i)),
        out_shape=jax.ShapeDtypeStruct((D, M), jnp.float32),
    )(counts, tableT)
    return outT.T


@jax.jit
def kernel(region_attention_table, region_map):
    counts = _make_hist_kernel()(region_map)
    return _scale(region_attention_table, counts)
